# Initial kernel scaffold; baseline (speedup 1.0000x reference)
import jax
import jax.numpy as jnp
from jax.experimental import pallas as pl


def kernel(z_e_x, iter, codebook):
    raise NotImplementedError("write your pallas kernel here")


# speedup vs baseline: 1.3456x; 1.3456x over previous
import functools

import jax
import jax.numpy as jnp
from jax import lax
from jax.experimental import pallas as pl
from jax.experimental.pallas import tpu as pltpu
from jax.experimental.pallas import tpu_sc as plsc

_K = 8192
_DIM = 32
_N = 16384

_BN = 256

_NC = 2
_NS = 16
_NW = _NC * _NS
_BPW = _N // _NW
_IDX_ROW = 128
_ROWS_PER_W = _BPW // _IDX_ROW


def _argmin_body(z_ref, cb_ref, out_ref, c2_ref):
    i = pl.program_id(0)

    @pl.when(i == 0)
    def _init():
        cb = cb_ref[...]
        c2_ref[...] = jnp.sum(cb * cb, axis=1, keepdims=True)

    z = z_ref[...]
    mT = lax.dot_general(
        cb_ref[...], z, (((1,), (1,)), ((), ())),
        preferred_element_type=jnp.float32)
    dT = c2_ref[...] - 2.0 * mT
    mn = jnp.min(dT, axis=0, keepdims=True)
    ids = lax.broadcasted_iota(jnp.int32, dT.shape, 0)
    idx = jnp.min(jnp.where(dT == mn, ids, _K), axis=0)
    out_ref[0, 0, :] = idx


def _pplx_body(cnt_ref, perp_ref, usage_ref):
    c = cnt_ref[0:1, :] + cnt_ref[1:2, :]
    total = jnp.sum(c)
    usage_ref[...] = jnp.sum(jnp.where(c >= 1.0, 1.0, 0.0),
                             axis=1, keepdims=True)
    prob = c / total
    perp_ref[...] = jnp.exp(-jnp.sum(prob * jnp.log(prob + 1e-7),
                                     axis=1, keepdims=True))


def _sc_gather_count(cb_hbm, idx_hbm, zeros_hbm, zq_hbm, cnt_hbm,
                     idx_v, ones_v, rows_v, cnt_sh, sem):
    c = lax.axis_index("c")
    s = lax.axis_index("s")
    wid = c * _NS + s

    pltpu.sync_copy(idx_hbm.at[pl.ds(wid * _ROWS_PER_W, _ROWS_PER_W)], idx_v)

    for i in range(_ROWS_PER_W):
        for j in range(_IDX_ROW // 16):
            ones_v[i, pl.ds(j * 16, 16)] = jnp.ones((16,), jnp.float32)

    cps = [
        pltpu.async_copy(cb_hbm.at[idx_v.at[i]],
                         rows_v.at[pl.ds(i * _IDX_ROW, _IDX_ROW)], sem)
        for i in range(_ROWS_PER_W)
    ]
    for cp in cps:
        cp.wait()
    pltpu.sync_copy(rows_v, zq_hbm.at[pl.ds(wid * _BPW, _BPW)])

    @pl.when(s == 0)
    def _zero():
        pltpu.sync_copy(zeros_hbm, cnt_sh)

    plsc.subcore_barrier()
    for i in range(_ROWS_PER_W):
        pltpu.sync_copy(ones_v.at[i], cnt_sh.at[idx_v.at[i]], add=True)
    plsc.subcore_barrier()

    @pl.when(s == 0)
    def _flush():
        pltpu.sync_copy(cnt_sh, cnt_hbm.at[c])


def _make_sc_call():
    mesh = plsc.VectorSubcoreMesh(core_axis_name="c", subcore_axis_name="s")
    return pl.kernel(
        _sc_gather_count,
        mesh=mesh,
        compiler_params=pltpu.CompilerParams(use_tc_tiling_on_sc=False),
        out_type=[
            jax.ShapeDtypeStruct((_N, _DIM), jnp.float32),
            jax.ShapeDtypeStruct((_NC, _K), jnp.float32),
        ],
        scratch_types=[
            pltpu.VMEM((_ROWS_PER_W, _IDX_ROW), jnp.int32),
            pltpu.VMEM((_ROWS_PER_W, _IDX_ROW), jnp.float32),
            pltpu.VMEM((_BPW, _DIM), jnp.float32),
            pltpu.VMEM_SHARED((_K,), jnp.float32),
            pltpu.SemaphoreType.DMA,
        ],
    )


def _argmin_call(z_e_x, codebook):
    nb = _N // _BN
    return pl.pallas_call(
        _argmin_body,
        grid=(nb,),
        in_specs=[
            pl.BlockSpec((_BN, _DIM), lambda i: (i, 0)),
            pl.BlockSpec((_K, _DIM), lambda i: (0, 0)),
        ],
        out_specs=pl.BlockSpec((1, 1, _BN), lambda i: (i, 0, 0)),
        out_shape=jax.ShapeDtypeStruct((nb, 1, _BN), jnp.int32),
        scratch_shapes=[pltpu.VMEM((_K, 1), jnp.float32)],
    )(z_e_x, codebook)


def _pplx_call(counts2):
    return pl.pallas_call(
        _pplx_body,
        out_shape=[
            jax.ShapeDtypeStruct((1, 1), jnp.float32),
            jax.ShapeDtypeStruct((1, 1), jnp.float32),
        ],
    )(counts2)


def kernel(z_e_x, iter, codebook):
    del iter
    code_idx = _argmin_call(z_e_x, codebook).reshape(_N)
    idx2d = code_idx.reshape(_N // _IDX_ROW, _IDX_ROW)
    zeros = jnp.zeros((_K,), jnp.float32)
    z_q_x, counts2 = _make_sc_call()(codebook, idx2d, zeros)
    perp, usage = _pplx_call(counts2)
    return (z_q_x, z_q_x, perp[0, 0], usage[0, 0])


# speedup vs baseline: 2.2494x; 1.6717x over previous
import functools

import jax
import jax.numpy as jnp
from jax import lax
from jax.experimental import pallas as pl
from jax.experimental.pallas import tpu as pltpu
from jax.experimental.pallas import tpu_sc as plsc

_K = 8192
_DIM = 32
_N = 16384

_BN = 1024

_NC = 2
_NS = 16
_NW = _NC * _NS
_BPW = _N // _NW
_IDX_ROW = 128
_ROWS_PER_W = _BPW // _IDX_ROW


_TILE_K = 4096
_CH = 8


def _argmin_body(z_ref, cb_ref, a_ref, c_ref, out_ref, cbb_ref):
    i = pl.program_id(0)

    @pl.when(i == 0)
    def _init():
        cbb_ref[...] = (-2.0 * cb_ref[...]).astype(jnp.bfloat16)

    zb = z_ref[...].astype(jnp.bfloat16)
    a = a_ref[0]
    af = jnp.broadcast_to(a, (_CH, _BN))
    rowpos = lax.broadcasted_iota(jnp.int32, (_CH, _BN), 0)
    cur = jnp.full((1, _BN), jnp.inf, jnp.float32)
    curi = jnp.zeros((1, _BN), jnp.int32)
    for t in range(_K // _TILE_K):
        mt = lax.dot_general(
            cbb_ref[pl.ds(t * _TILE_K, _TILE_K), :], zb,
            (((1,), (1,)), ((), ())),
            preferred_element_type=jnp.float32)
        acc = jnp.full((_CH, _BN), jnp.inf, jnp.float32)
        acci = jnp.zeros((_CH, _BN), jnp.int32)
        for r in range(_TILE_K // _CH):
            dtc = ((af + mt[r * _CH:(r + 1) * _CH, :])
                   + c_ref[pl.ds(t * _TILE_K + r * _CH, _CH), :])
            m = dtc < acc
            acc = jnp.minimum(acc, dtc)
            acci = jnp.where(m, r, acci)
        rows = acci * _CH + rowpos + t * _TILE_K
        tmin = jnp.min(acc, axis=0, keepdims=True)
        targ = jnp.min(jnp.where(acc == tmin, rows, _K), axis=0, keepdims=True)
        take = tmin < cur
        cur = jnp.where(take, tmin.astype(jnp.bfloat16).astype(jnp.float32), cur)
        curi = jnp.where(take, targ, curi)
    out_ref[0, 0, :] = curi[0]


def _pplx_body(cnt_ref, perp_ref, usage_ref):
    c = cnt_ref[0:1, :] + cnt_ref[1:2, :]
    total = jnp.sum(c)
    usage_ref[...] = jnp.sum(jnp.where(c >= 1.0, 1.0, 0.0),
                             axis=1, keepdims=True)
    prob = c / total
    perp_ref[...] = jnp.exp(-jnp.sum(prob * jnp.log(prob + 1e-7),
                                     axis=1, keepdims=True))


def _sc_gather_count(cb_hbm, idx_hbm, zeros_hbm, zq_hbm, cnt_hbm,
                     idx_v, ones_v, rows_v, cnt_sh, sem):
    c = lax.axis_index("c")
    s = lax.axis_index("s")
    wid = c * _NS + s

    pltpu.sync_copy(idx_hbm.at[pl.ds(wid * _ROWS_PER_W, _ROWS_PER_W)], idx_v)

    for i in range(_ROWS_PER_W):
        for j in range(_IDX_ROW // 16):
            ones_v[i, pl.ds(j * 16, 16)] = jnp.ones((16,), jnp.float32)

    cps = [
        pltpu.async_copy(cb_hbm.at[idx_v.at[i]],
                         rows_v.at[pl.ds(i * _IDX_ROW, _IDX_ROW)], sem)
        for i in range(_ROWS_PER_W)
    ]
    for cp in cps:
        cp.wait()
    pltpu.sync_copy(rows_v, zq_hbm.at[pl.ds(wid * _BPW, _BPW)])

    @pl.when(s == 0)
    def _zero():
        pltpu.sync_copy(zeros_hbm, cnt_sh)

    plsc.subcore_barrier()
    for i in range(_ROWS_PER_W):
        pltpu.sync_copy(ones_v.at[i], cnt_sh.at[idx_v.at[i]], add=True)
    plsc.subcore_barrier()

    @pl.when(s == 0)
    def _flush():
        pltpu.sync_copy(cnt_sh, cnt_hbm.at[c])


def _make_sc_call():
    mesh = plsc.VectorSubcoreMesh(core_axis_name="c", subcore_axis_name="s")
    return pl.kernel(
        _sc_gather_count,
        mesh=mesh,
        compiler_params=pltpu.CompilerParams(use_tc_tiling_on_sc=False),
        out_type=[
            jax.ShapeDtypeStruct((_N, _DIM), jnp.float32),
            jax.ShapeDtypeStruct((_NC, _K), jnp.float32),
        ],
        scratch_types=[
            pltpu.VMEM((_ROWS_PER_W, _IDX_ROW), jnp.int32),
            pltpu.VMEM((_ROWS_PER_W, _IDX_ROW), jnp.float32),
            pltpu.VMEM((_BPW, _DIM), jnp.float32),
            pltpu.VMEM_SHARED((_K,), jnp.float32),
            pltpu.SemaphoreType.DMA,
        ],
    )


def _argmin_call(z_e_x, codebook):
    nb = _N // _BN
    a = jnp.sum(z_e_x ** 2, axis=-1).reshape(nb, 1, _BN)
    c = jnp.sum(codebook ** 2, axis=1).reshape(_K, 1)
    return pl.pallas_call(
        _argmin_body,
        grid=(nb,),
        in_specs=[
            pl.BlockSpec((_BN, _DIM), lambda i: (i, 0)),
            pl.BlockSpec((_K, _DIM), lambda i: (0, 0)),
            pl.BlockSpec((1, 1, _BN), lambda i: (i, 0, 0)),
            pl.BlockSpec((_K, 1), lambda i: (0, 0)),
        ],
        out_specs=pl.BlockSpec((1, 1, _BN), lambda i: (i, 0, 0)),
        out_shape=jax.ShapeDtypeStruct((nb, 1, _BN), jnp.int32),
        scratch_shapes=[pltpu.VMEM((_K, _DIM), jnp.bfloat16)],
    )(z_e_x, codebook, a, c)


def _pplx_call(counts2):
    return pl.pallas_call(
        _pplx_body,
        out_shape=[
            jax.ShapeDtypeStruct((1, 1), jnp.float32),
            jax.ShapeDtypeStruct((1, 1), jnp.float32),
        ],
    )(counts2)


def kernel(z_e_x, iter, codebook):
    del iter
    code_idx = _argmin_call(z_e_x, codebook).reshape(_N)
    idx2d = code_idx.reshape(_N // _IDX_ROW, _IDX_ROW)
    zeros = jnp.zeros((_K,), jnp.float32)
    z_q_x, counts2 = _make_sc_call()(codebook, idx2d, zeros)
    perp, usage = _pplx_call(counts2)
    return (z_q_x, z_q_x, perp[0, 0], usage[0, 0])
